# baseline (speedup 1.0000x reference)
import jax
import jax.numpy as jnp
from jax import lax
from jax.experimental import pallas as pl
from jax.experimental.pallas import tpu as pltpu
from jax.experimental.pallas import tpu_sc as plsc

O_ = 10000
O_PAD = 10240
T_ = 320000
D_ = 128
H_ = 512
DOUT_ = 128

NC = 2
NS = 16
NW = NC * NS

NGRP = T_ // 128
ZROWS = 80

BT = 3200
NHALF = 4
TH = T_ // NHALF

_bf16 = jnp.bfloat16
_f32 = jnp.float32


def _gather_body(obj_ref, sidx_ref, oidx_ref, cs_ref, co_ref, cnt_ref,
                 idx_s3, idx_o3, rows_s3, rows_o3, ones_v, zc,
                 lsem, gsem, wsem, cacc):
    c = lax.axis_index("c")
    s = lax.axis_index("s")
    wid = s * NC + c
    ngrp = sidx_ref.shape[0]
    n_my = (ngrp - wid + NW - 1) // NW

    zero16 = jnp.zeros((16,), _f32)
    one16 = jnp.ones((16,), _f32)
    for j in range(8):
        ones_v[pl.ds(j * 16, 16)] = one16
    def zcrow(i, carry):
        zc[pl.ds(i * 16, 16)] = zero16
        return carry
    lax.fori_loop(0, 640 // 16, zcrow, 0)
    pltpu.sync_copy(zc, cacc.at[pl.ds(s * 640, 640)])
    plsc.subcore_barrier()

    def start_load(i, sl):
        g = wid + i * NW
        pltpu.async_copy(sidx_ref.at[pl.ds(g, 1)],
                         idx_s3.at[pl.ds(sl, 1)], lsem)
        pltpu.async_copy(oidx_ref.at[pl.ds(g, 1)],
                         idx_o3.at[pl.ds(sl, 1)], lsem)

    def drain_writeouts():
        pltpu.make_async_copy(cs_ref.at[pl.ds(0, 128)],
                              rows_s3.at[pl.ds(0, 128)], wsem).wait()
        pltpu.make_async_copy(co_ref.at[pl.ds(0, 128)],
                              rows_o3.at[pl.ds(0, 128)], wsem).wait()

    def wave(w, carry):
        for b in range(3):
            i = w * 3 + b
            @pl.when(jnp.logical_and(i >= 2, i < n_my))
            def _():
                drain_writeouts()
            @pl.when(i + 1 < n_my)
            def _():
                start_load(i + 1, (b + 1) % 3)
            @pl.when(i < n_my)
            def _():
                g = wid + i * NW
                base = g * 128
                pltpu.make_async_copy(sidx_ref.at[pl.ds(0, 1)],
                                      idx_s3.at[pl.ds(b, 1)], lsem).wait()
                pltpu.make_async_copy(oidx_ref.at[pl.ds(0, 1)],
                                      idx_o3.at[pl.ds(b, 1)], lsem).wait()
                cp_s = pltpu.async_copy(
                    obj_ref.at[idx_s3.at[b]],
                    rows_s3.at[pl.ds(b * 128, 128)], gsem)
                cp_o = pltpu.async_copy(
                    obj_ref.at[idx_o3.at[b]],
                    rows_o3.at[pl.ds(b * 128, 128)], gsem)
                pltpu.sync_copy(ones_v, cacc.at[idx_s3.at[b]], add=True)
                pltpu.sync_copy(ones_v, cacc.at[idx_o3.at[b]], add=True)
                cp_s.wait()
                cp_o.wait()
                pltpu.async_copy(rows_s3.at[pl.ds(b * 128, 128)],
                                 cs_ref.at[pl.ds(base, 128)], wsem)
                pltpu.async_copy(rows_o3.at[pl.ds(b * 128, 128)],
                                 co_ref.at[pl.ds(base, 128)], wsem)
        return carry

    start_load(0, 0)
    lax.fori_loop(0, (n_my + 2) // 3, wave, 0)
    drain_writeouts()
    drain_writeouts()
    plsc.subcore_barrier()
    pltpu.sync_copy(cacc.at[pl.ds(s * 640, 640)],
                    cnt_ref.at[c].at[pl.ds(s * 640, 640)])


def _gather(obj_vecs, sidx2, oidx2):
    tq = sidx2.shape[0] * 128
    f = pl.kernel(
        _gather_body,
        out_type=(jax.ShapeDtypeStruct((tq, D_), _f32),
                  jax.ShapeDtypeStruct((tq, D_), _f32),
                  jax.ShapeDtypeStruct((2, O_PAD), _f32)),
        mesh=plsc.VectorSubcoreMesh(core_axis_name="c", subcore_axis_name="s"),
        scratch_types=[
            pltpu.VMEM((3, 128), jnp.int32),
            pltpu.VMEM((3, 128), jnp.int32),
            pltpu.VMEM((3 * 128, D_), _f32),
            pltpu.VMEM((3 * 128, D_), _f32),
            pltpu.VMEM((128,), _f32),
            pltpu.VMEM((640,), _f32),
            pltpu.SemaphoreType.DMA,
            pltpu.SemaphoreType.DMA,
            pltpu.SemaphoreType.DMA,
            pltpu.VMEM_SHARED((O_PAD,), _f32),
        ],
    )
    return f(obj_vecs, sidx2, oidx2)


def _net1_body(cs_ref, pred_ref, co_ref, w1as_ref, w1ap_ref, w1ao_ref,
               b1a_ref, w1b_ref, b1b_ref, ns_ref, np_ref, no_ref):
    sb = cs_ref[...].astype(_bf16)
    pb = pred_ref[...].astype(_bf16)
    ob = co_ref[...].astype(_bf16)
    h = jnp.dot(sb, w1as_ref[...], preferred_element_type=_f32)
    h = h + jnp.dot(pb, w1ap_ref[...], preferred_element_type=_f32)
    h = h + jnp.dot(ob, w1ao_ref[...], preferred_element_type=_f32)
    h = jnp.maximum(h + b1a_ref[...], 0.0).astype(_bf16)
    t = jnp.dot(h, w1b_ref[...], preferred_element_type=_f32) + b1b_ref[...]
    t = jnp.maximum(t, 0.0)
    np_ref[...] = t[:, H_:H_ + DOUT_]
    for k in range(4):
        ns_ref[k, :, :] = t[:, k * 128:(k + 1) * 128]
        no_ref[k, :, :] = t[:, H_ + DOUT_ + k * 128:H_ + DOUT_ + (k + 1) * 128]


def _net1(cs, pred, co, w1as, w1ap, w1ao, b1a, w1b, b1b, blk0):
    nblk = TH // BT
    return pl.pallas_call(
        _net1_body,
        grid=(nblk,),
        in_specs=[
            pl.BlockSpec((BT, D_), lambda i: (i, 0)),
            pl.BlockSpec((BT, D_), lambda i: (i + blk0, 0)),
            pl.BlockSpec((BT, D_), lambda i: (i, 0)),
            pl.BlockSpec((D_, H_), lambda i: (0, 0)),
            pl.BlockSpec((D_, H_), lambda i: (0, 0)),
            pl.BlockSpec((D_, H_), lambda i: (0, 0)),
            pl.BlockSpec((1, H_), lambda i: (0, 0)),
            pl.BlockSpec((H_, 2 * H_ + DOUT_), lambda i: (0, 0)),
            pl.BlockSpec((1, 2 * H_ + DOUT_), lambda i: (0, 0)),
        ],
        out_specs=[
            pl.BlockSpec((4, BT, 128), lambda i: (0, i, 0)),
            pl.BlockSpec((BT, DOUT_), lambda i: (i, 0)),
            pl.BlockSpec((4, BT, 128), lambda i: (0, i, 0)),
        ],
        out_shape=[
            jax.ShapeDtypeStruct((4, TH, 128), _f32),
            jax.ShapeDtypeStruct((TH, DOUT_), _f32),
            jax.ShapeDtypeStruct((4, TH, 128), _f32),
        ],
    )(cs, pred, co, w1as, w1ap, w1ao, b1a, w1b, b1b)


def _scatter_body_impl(ns_ref, no_ref, sidx_ref, oidx_ref, init_ref,
                       pooled_ref, idx3, rows3, lsem, ssem, acc):
    c = lax.axis_index("c")
    s = lax.axis_index("s")
    ngrp = sidx_ref.shape[0]

    for cc in range(2):
        k = 2 * c + cc

        if init_ref is None:
            zero16 = jnp.zeros((16,), _f32)
            def zrow(i, carry):
                for j in range(8):
                    rows3[i, pl.ds(j * 16, 16)] = zero16
                return carry
            lax.fori_loop(0, ZROWS, zrow, 0)
            zbase = s * 640
            nz = jnp.where(s < 15, 8, 5)
            def zero_acc(j, carry):
                pltpu.sync_copy(rows3.at[pl.ds(0, ZROWS)],
                                acc.at[pl.ds(zbase + j * ZROWS, ZROWS)])
                return carry
            lax.fori_loop(0, nz, zero_acc, 0)
        else:
            @pl.when(s < 15)
            def _():
                pltpu.sync_copy(init_ref.at[k].at[pl.ds(s * 640, 640)],
                                acc.at[pl.ds(s * 640, 640)])
            @pl.when(s == 15)
            def _():
                pltpu.sync_copy(init_ref.at[k].at[pl.ds(9600, 400)],
                                acc.at[pl.ds(9600, 400)])
        plsc.subcore_barrier()

        for a in range(2):
            src_ref = ns_ref if a == 0 else no_ref
            idx_ref = sidx_ref if a == 0 else oidx_ref
            n_my = (ngrp - s + NS - 1) // NS

            def start_load(i, sl):
                g = s + i * NS
                pltpu.async_copy(idx_ref.at[pl.ds(g, 1)],
                                 idx3.at[pl.ds(sl, 1)], lsem)
                pltpu.async_copy(src_ref.at[k].at[pl.ds(g * 128, 128)],
                                 rows3.at[pl.ds(sl * 128, 128)], lsem)

            def drain_scatter():
                pltpu.make_async_copy(src_ref.at[0].at[pl.ds(0, 128)],
                                      rows3.at[pl.ds(0, 128)], ssem).wait()

            def wave(w, carry):
                for b in range(3):
                    i = w * 3 + b
                    @pl.when(jnp.logical_and(i >= 2, i < n_my))
                    def _():
                        drain_scatter()
                    @pl.when(i + 1 < n_my)
                    def _():
                        start_load(i + 1, (b + 1) % 3)
                    @pl.when(i < n_my)
                    def _():
                        pltpu.make_async_copy(idx_ref.at[pl.ds(0, 1)],
                                              idx3.at[pl.ds(b, 1)],
                                              lsem).wait()
                        pltpu.make_async_copy(src_ref.at[0].at[pl.ds(0, 128)],
                                              rows3.at[pl.ds(b * 128, 128)],
                                              lsem).wait()
                        pltpu.async_copy(rows3.at[pl.ds(b * 128, 128)],
                                         acc.at[idx3.at[b]], ssem, add=True)
                return carry

            start_load(0, 0)
            lax.fori_loop(0, (n_my + 2) // 3, wave, 0)
            drain_scatter()
            drain_scatter()

        plsc.subcore_barrier()
        @pl.when(s < 10)
        def _():
            pltpu.sync_copy(
                acc.at[pl.ds(s * 1000, 1000)],
                pooled_ref.at[k].at[pl.ds(s * 1000, 1000)])
        plsc.subcore_barrier()


def _scatter(ns4, no4, sidx2, oidx2, init4=None):
    if init4 is None:
        def body(ns, no, si, oi, pooled, *scratch):
            _scatter_body_impl(ns, no, si, oi, None, pooled, *scratch)
    else:
        def body(ns, no, si, oi, init, pooled, *scratch):
            _scatter_body_impl(ns, no, si, oi, init, pooled, *scratch)
    f = pl.kernel(
        body,
        out_type=jax.ShapeDtypeStruct((4, O_, 128), _f32),
        mesh=plsc.VectorSubcoreMesh(core_axis_name="c", subcore_axis_name="s"),
        scratch_types=[
            pltpu.VMEM((3, 128), jnp.int32),
            pltpu.VMEM((3 * 128, 128), _f32),
            pltpu.SemaphoreType.DMA,
            pltpu.SemaphoreType.DMA,
            pltpu.VMEM_SHARED((O_, 128), _f32),
        ],
    )
    if init4 is None:
        return f(ns4, no4, sidx2, oidx2)
    return f(ns4, no4, sidx2, oidx2, init4)


def _net2_body(p4_ref, cnt_ref, w2a_ref, b2a_ref, w2b_ref, b2b_ref, out_ref):
    cnt = cnt_ref[0, :]
    for r in range(1, cnt_ref.shape[0]):
        cnt = cnt + cnt_ref[r, :]
    inv = 1.0 / jnp.maximum(cnt, 1.0)
    pooled = jnp.concatenate([p4_ref[k] for k in range(4)], axis=1)
    pooled = (pooled * inv[:, None]).astype(_bf16)
    h = jnp.maximum(
        jnp.dot(pooled, w2a_ref[...], preferred_element_type=_f32)
        + b2a_ref[...], 0.0).astype(_bf16)
    out_ref[...] = jnp.maximum(
        jnp.dot(h, w2b_ref[...], preferred_element_type=_f32)
        + b2b_ref[...], 0.0)


def _net2(pooled4, counts2, w2a, b2a, w2b, b2b):
    return pl.pallas_call(
        _net2_body,
        out_shape=jax.ShapeDtypeStruct((O_, DOUT_), _f32),
    )(pooled4, counts2, w2a, b2a, w2b, b2b)


def kernel(obj_vecs, pred_vecs, edges, W1a, b1a, W1b, b1b, W2a, b2a, W2b, b2b):
    s_idx = edges[:, 0]
    o_idx = edges[:, 1]
    sidx2 = s_idx.reshape(T_ // 128, 128)
    oidx2 = o_idx.reshape(T_ // 128, 128)
    w1as = W1a[:D_].astype(_bf16)
    w1ap = W1a[D_:2 * D_].astype(_bf16)
    w1ao = W1a[2 * D_:].astype(_bf16)
    w1b = W1b.astype(_bf16)
    b1a2 = b1a.reshape(1, H_)
    b1b2 = b1b.reshape(1, 2 * H_ + DOUT_)

    gh = TH // 128
    gathered = []
    for h in range(NHALF):
        gathered.append(_gather(obj_vecs,
                                sidx2[h * gh:(h + 1) * gh],
                                oidx2[h * gh:(h + 1) * gh]))
    pooled4 = None
    new_ps = []
    for h in range(NHALF):
        csh, coh, _ = gathered[h]
        ns4h, nph, no4h = _net1(csh, pred_vecs, coh, w1as, w1ap, w1ao,
                                b1a2, w1b, b1b2, h * (TH // BT))
        new_ps.append(nph)
        pooled4 = _scatter(ns4h, no4h,
                           sidx2[h * gh:(h + 1) * gh],
                           oidx2[h * gh:(h + 1) * gh], pooled4)
    new_p = jnp.concatenate(new_ps, axis=0)
    cntall = jnp.concatenate([g[2] for g in gathered], axis=0)[:, :O_]

    new_obj = _net2(pooled4, cntall, W2a.astype(_bf16), b2a.reshape(1, H_),
                    W2b.astype(_bf16), b2b.reshape(1, DOUT_))
    return new_obj, new_p


# speedup vs baseline: 1.0038x; 1.0038x over previous
import jax
import jax.numpy as jnp
from jax import lax
from jax.experimental import pallas as pl
from jax.experimental.pallas import tpu as pltpu
from jax.experimental.pallas import tpu_sc as plsc

O_ = 10000
O_PAD = 10240
T_ = 320000
D_ = 128
H_ = 512
DOUT_ = 128

NC = 2
NS = 16
NW = NC * NS

NGRP = T_ // 128
ZROWS = 80

BT = 3200
NHALF = 4
TH = T_ // NHALF

_bf16 = jnp.bfloat16
_f32 = jnp.float32


def _gather_body(obj_ref, sidx_ref, oidx_ref, cs_ref, co_ref, cnt_ref,
                 idx_s3, idx_o3, rows_s3, rows_o3, ones_v, zc,
                 lsem, gsem, wsem, cacc):
    c = lax.axis_index("c")
    s = lax.axis_index("s")
    wid = s * NC + c
    ngrp = sidx_ref.shape[0]
    n_my = (ngrp - wid + NW - 1) // NW

    zero16 = jnp.zeros((16,), _f32)
    one16 = jnp.ones((16,), _f32)
    for j in range(8):
        ones_v[pl.ds(j * 16, 16)] = one16
    def zcrow(i, carry):
        zc[pl.ds(i * 16, 16)] = zero16
        return carry
    lax.fori_loop(0, 640 // 16, zcrow, 0)
    pltpu.sync_copy(zc, cacc.at[pl.ds(s * 640, 640)])
    plsc.subcore_barrier()

    def start_load(i, sl):
        g = wid + i * NW
        pltpu.async_copy(sidx_ref.at[pl.ds(g, 1)],
                         idx_s3.at[pl.ds(sl, 1)], lsem)
        pltpu.async_copy(oidx_ref.at[pl.ds(g, 1)],
                         idx_o3.at[pl.ds(sl, 1)], lsem)

    def drain_writeouts():
        pltpu.make_async_copy(cs_ref.at[pl.ds(0, 128)],
                              rows_s3.at[pl.ds(0, 128)], wsem).wait()
        pltpu.make_async_copy(co_ref.at[pl.ds(0, 128)],
                              rows_o3.at[pl.ds(0, 128)], wsem).wait()

    def wave(w, carry):
        for b in range(3):
            i = w * 3 + b
            @pl.when(jnp.logical_and(i >= 2, i < n_my))
            def _():
                drain_writeouts()
            @pl.when(i + 1 < n_my)
            def _():
                start_load(i + 1, (b + 1) % 3)
            @pl.when(i < n_my)
            def _():
                g = wid + i * NW
                base = g * 128
                pltpu.make_async_copy(sidx_ref.at[pl.ds(0, 1)],
                                      idx_s3.at[pl.ds(b, 1)], lsem).wait()
                pltpu.make_async_copy(oidx_ref.at[pl.ds(0, 1)],
                                      idx_o3.at[pl.ds(b, 1)], lsem).wait()
                cp_s = pltpu.async_copy(
                    obj_ref.at[idx_s3.at[b]],
                    rows_s3.at[pl.ds(b * 128, 128)], gsem)
                cp_o = pltpu.async_copy(
                    obj_ref.at[idx_o3.at[b]],
                    rows_o3.at[pl.ds(b * 128, 128)], gsem)
                pltpu.sync_copy(ones_v, cacc.at[idx_s3.at[b]], add=True)
                pltpu.sync_copy(ones_v, cacc.at[idx_o3.at[b]], add=True)
                cp_s.wait()
                cp_o.wait()
                pltpu.async_copy(rows_s3.at[pl.ds(b * 128, 128)],
                                 cs_ref.at[pl.ds(base, 128)], wsem)
                pltpu.async_copy(rows_o3.at[pl.ds(b * 128, 128)],
                                 co_ref.at[pl.ds(base, 128)], wsem)
        return carry

    start_load(0, 0)
    lax.fori_loop(0, (n_my + 2) // 3, wave, 0)
    drain_writeouts()
    drain_writeouts()
    plsc.subcore_barrier()
    pltpu.sync_copy(cacc.at[pl.ds(s * 640, 640)],
                    cnt_ref.at[c].at[pl.ds(s * 640, 640)])


def _gather(obj_vecs, sidx2, oidx2):
    tq = sidx2.shape[0] * 128
    f = pl.kernel(
        _gather_body,
        out_type=(jax.ShapeDtypeStruct((tq, D_), _f32),
                  jax.ShapeDtypeStruct((tq, D_), _f32),
                  jax.ShapeDtypeStruct((2, O_PAD), _f32)),
        mesh=plsc.VectorSubcoreMesh(core_axis_name="c", subcore_axis_name="s"),
        scratch_types=[
            pltpu.VMEM((3, 128), jnp.int32),
            pltpu.VMEM((3, 128), jnp.int32),
            pltpu.VMEM((3 * 128, D_), _f32),
            pltpu.VMEM((3 * 128, D_), _f32),
            pltpu.VMEM((128,), _f32),
            pltpu.VMEM((640,), _f32),
            pltpu.SemaphoreType.DMA,
            pltpu.SemaphoreType.DMA,
            pltpu.SemaphoreType.DMA,
            pltpu.VMEM_SHARED((O_PAD,), _f32),
        ],
    )
    return f(obj_vecs, sidx2, oidx2)


def _net1_body(cs_ref, pred_ref, co_ref, w1as_ref, w1ap_ref, w1ao_ref,
               b1a_ref, w1b_ref, b1b_ref, ns_ref, np_ref, no_ref):
    sb = cs_ref[...].astype(_bf16)
    pb = pred_ref[...].astype(_bf16)
    ob = co_ref[...].astype(_bf16)
    h = jnp.dot(sb, w1as_ref[...], preferred_element_type=_f32)
    h = h + jnp.dot(pb, w1ap_ref[...], preferred_element_type=_f32)
    h = h + jnp.dot(ob, w1ao_ref[...], preferred_element_type=_f32)
    h = jnp.maximum(h + b1a_ref[...], 0.0).astype(_bf16)
    t = jnp.dot(h, w1b_ref[...], preferred_element_type=_f32) + b1b_ref[...]
    t = jnp.maximum(t, 0.0)
    np_ref[...] = t[:, H_:H_ + DOUT_]
    for k in range(4):
        ns_ref[k, :, :] = t[:, k * 128:(k + 1) * 128]
        no_ref[k, :, :] = t[:, H_ + DOUT_ + k * 128:H_ + DOUT_ + (k + 1) * 128]


def _net1(cs, pred, co, w1as, w1ap, w1ao, b1a, w1b, b1b, blk0):
    nblk = TH // BT
    return pl.pallas_call(
        _net1_body,
        grid=(nblk,),
        in_specs=[
            pl.BlockSpec((BT, D_), lambda i: (i + blk0, 0)),
            pl.BlockSpec((BT, D_), lambda i: (i + blk0, 0)),
            pl.BlockSpec((BT, D_), lambda i: (i + blk0, 0)),
            pl.BlockSpec((D_, H_), lambda i: (0, 0)),
            pl.BlockSpec((D_, H_), lambda i: (0, 0)),
            pl.BlockSpec((D_, H_), lambda i: (0, 0)),
            pl.BlockSpec((1, H_), lambda i: (0, 0)),
            pl.BlockSpec((H_, 2 * H_ + DOUT_), lambda i: (0, 0)),
            pl.BlockSpec((1, 2 * H_ + DOUT_), lambda i: (0, 0)),
        ],
        out_specs=[
            pl.BlockSpec((4, BT, 128), lambda i: (0, i, 0)),
            pl.BlockSpec((BT, DOUT_), lambda i: (i, 0)),
            pl.BlockSpec((4, BT, 128), lambda i: (0, i, 0)),
        ],
        out_shape=[
            jax.ShapeDtypeStruct((4, TH, 128), _f32),
            jax.ShapeDtypeStruct((TH, DOUT_), _f32),
            jax.ShapeDtypeStruct((4, TH, 128), _f32),
        ],
    )(cs, pred, co, w1as, w1ap, w1ao, b1a, w1b, b1b)


def _scatter_body_impl(ns_ref, no_ref, sidx_ref, oidx_ref, init_ref,
                       pooled_ref, idx3, rows3, lsem, ssem, acc):
    c = lax.axis_index("c")
    s = lax.axis_index("s")
    ngrp = sidx_ref.shape[0]

    for cc in range(2):
        k = 2 * c + cc

        if init_ref is None:
            zero16 = jnp.zeros((16,), _f32)
            def zrow(i, carry):
                for j in range(8):
                    rows3[i, pl.ds(j * 16, 16)] = zero16
                return carry
            lax.fori_loop(0, ZROWS, zrow, 0)
            zbase = s * 640
            nz = jnp.where(s < 15, 8, 5)
            def zero_acc(j, carry):
                pltpu.sync_copy(rows3.at[pl.ds(0, ZROWS)],
                                acc.at[pl.ds(zbase + j * ZROWS, ZROWS)])
                return carry
            lax.fori_loop(0, nz, zero_acc, 0)
        else:
            @pl.when(s < 15)
            def _():
                pltpu.sync_copy(init_ref.at[k].at[pl.ds(s * 640, 640)],
                                acc.at[pl.ds(s * 640, 640)])
            @pl.when(s == 15)
            def _():
                pltpu.sync_copy(init_ref.at[k].at[pl.ds(9600, 400)],
                                acc.at[pl.ds(9600, 400)])
        plsc.subcore_barrier()

        for a in range(2):
            src_ref = ns_ref if a == 0 else no_ref
            idx_ref = sidx_ref if a == 0 else oidx_ref
            n_my = (ngrp - s + NS - 1) // NS

            def start_load(i, sl):
                g = s + i * NS
                pltpu.async_copy(idx_ref.at[pl.ds(g, 1)],
                                 idx3.at[pl.ds(sl, 1)], lsem)
                pltpu.async_copy(src_ref.at[k].at[pl.ds(g * 128, 128)],
                                 rows3.at[pl.ds(sl * 128, 128)], lsem)

            def drain_scatter():
                pltpu.make_async_copy(src_ref.at[0].at[pl.ds(0, 128)],
                                      rows3.at[pl.ds(0, 128)], ssem).wait()

            def wave(w, carry):
                for b in range(3):
                    i = w * 3 + b
                    @pl.when(jnp.logical_and(i >= 2, i < n_my))
                    def _():
                        drain_scatter()
                    @pl.when(i + 1 < n_my)
                    def _():
                        start_load(i + 1, (b + 1) % 3)
                    @pl.when(i < n_my)
                    def _():
                        pltpu.make_async_copy(idx_ref.at[pl.ds(0, 1)],
                                              idx3.at[pl.ds(b, 1)],
                                              lsem).wait()
                        pltpu.make_async_copy(src_ref.at[0].at[pl.ds(0, 128)],
                                              rows3.at[pl.ds(b * 128, 128)],
                                              lsem).wait()
                        pltpu.async_copy(rows3.at[pl.ds(b * 128, 128)],
                                         acc.at[idx3.at[b]], ssem, add=True)
                return carry

            start_load(0, 0)
            lax.fori_loop(0, (n_my + 2) // 3, wave, 0)
            drain_scatter()
            drain_scatter()

        plsc.subcore_barrier()
        @pl.when(s < 10)
        def _():
            pltpu.sync_copy(
                acc.at[pl.ds(s * 1000, 1000)],
                pooled_ref.at[k].at[pl.ds(s * 1000, 1000)])
        plsc.subcore_barrier()


def _scatter(ns4, no4, sidx2, oidx2, init4=None):
    if init4 is None:
        def body(ns, no, si, oi, pooled, *scratch):
            _scatter_body_impl(ns, no, si, oi, None, pooled, *scratch)
    else:
        def body(ns, no, si, oi, init, pooled, *scratch):
            _scatter_body_impl(ns, no, si, oi, init, pooled, *scratch)
    f = pl.kernel(
        body,
        out_type=jax.ShapeDtypeStruct((4, O_, 128), _f32),
        mesh=plsc.VectorSubcoreMesh(core_axis_name="c", subcore_axis_name="s"),
        scratch_types=[
            pltpu.VMEM((3, 128), jnp.int32),
            pltpu.VMEM((3 * 128, 128), _f32),
            pltpu.SemaphoreType.DMA,
            pltpu.SemaphoreType.DMA,
            pltpu.VMEM_SHARED((O_, 128), _f32),
        ],
    )
    if init4 is None:
        return f(ns4, no4, sidx2, oidx2)
    return f(ns4, no4, sidx2, oidx2, init4)


def _net2_body(p4_ref, cnt_ref, w2a_ref, b2a_ref, w2b_ref, b2b_ref, out_ref):
    cnt = cnt_ref[0, :]
    for r in range(1, cnt_ref.shape[0]):
        cnt = cnt + cnt_ref[r, :]
    inv = 1.0 / jnp.maximum(cnt, 1.0)
    pooled = jnp.concatenate([p4_ref[k] for k in range(4)], axis=1)
    pooled = (pooled * inv[:, None]).astype(_bf16)
    h = jnp.maximum(
        jnp.dot(pooled, w2a_ref[...], preferred_element_type=_f32)
        + b2a_ref[...], 0.0).astype(_bf16)
    out_ref[...] = jnp.maximum(
        jnp.dot(h, w2b_ref[...], preferred_element_type=_f32)
        + b2b_ref[...], 0.0)


def _net2(pooled4, counts2, w2a, b2a, w2b, b2b):
    return pl.pallas_call(
        _net2_body,
        out_shape=jax.ShapeDtypeStruct((O_, DOUT_), _f32),
    )(pooled4, counts2, w2a, b2a, w2b, b2b)


def kernel(obj_vecs, pred_vecs, edges, W1a, b1a, W1b, b1b, W2a, b2a, W2b, b2b):
    s_idx = edges[:, 0]
    o_idx = edges[:, 1]
    sidx2 = s_idx.reshape(T_ // 128, 128)
    oidx2 = o_idx.reshape(T_ // 128, 128)
    w1as = W1a[:D_].astype(_bf16)
    w1ap = W1a[D_:2 * D_].astype(_bf16)
    w1ao = W1a[2 * D_:].astype(_bf16)
    w1b = W1b.astype(_bf16)
    b1a2 = b1a.reshape(1, H_)
    b1b2 = b1b.reshape(1, 2 * H_ + DOUT_)

    gh = TH // 128
    cs, co, cnt2p = _gather(obj_vecs, sidx2, oidx2)
    cntall = cnt2p[:, :O_]
    pooled4 = None
    new_ps = []
    for h in range(NHALF):
        ns4h, nph, no4h = _net1(cs, pred_vecs, co, w1as, w1ap, w1ao,
                                b1a2, w1b, b1b2, h * (TH // BT))
        new_ps.append(nph)
        pooled4 = _scatter(ns4h, no4h,
                           sidx2[h * gh:(h + 1) * gh],
                           oidx2[h * gh:(h + 1) * gh], pooled4)
    new_p = jnp.concatenate(new_ps, axis=0)

    new_obj = _net2(pooled4, cntall, W2a.astype(_bf16), b2a.reshape(1, H_),
                    W2b.astype(_bf16), b2b.reshape(1, DOUT_))
    return new_obj, new_p
